# baseline (speedup 1.0000x reference)
import functools

import jax
import jax.numpy as jnp
from jax.experimental import pallas as pl
from jax.experimental.pallas import tpu as pltpu

_SLOPE = 0.05


def _se_kernel(x_ref, w1t_ref, b1_ref, w2t_ref, b2_ref, o_ref, *,
               hw, pair):
    x = x_ref[...]
    if pair == 2:
        lane = jax.lax.broadcasted_iota(jnp.int32, x.shape, 2)
        first = lane < hw
        pe = jnp.sum(jnp.where(first, x, 0.0), axis=2)
        po = jnp.sum(jnp.where(first, 0.0, x), axis=2)
        pooled = jnp.concatenate([pe, po], axis=1) * (1.0 / hw)
    else:
        pooled = jnp.sum(x, axis=2) * (1.0 / hw)
    h = jnp.dot(pooled, w1t_ref[...],
                preferred_element_type=jnp.float32) + b1_ref[...]
    h = jnp.maximum(h, 0.0) + _SLOPE * jnp.minimum(h, 0.0)
    g = jnp.dot(h, w2t_ref[...],
                preferred_element_type=jnp.float32) + b2_ref[...]
    s = jax.nn.sigmoid(g)
    if pair == 2:
        half = s.shape[1] // 2
        se = s[:, :half, None]
        so = s[:, half:, None]
        o_ref[...] = x * jnp.where(first, se, so)
    else:
        o_ref[...] = x * s[:, :, None]


def kernel(x_nchw, w1, b1, w2, b2):
    B, C, H, W = x_nchw.shape
    HW = H * W
    pair = 2 if C % 2 == 0 else 1
    rows = C // pair
    lanes = pair * HW
    xv = x_nchw.reshape(B, rows, lanes)

    if pair == 2:
        idx = jnp.concatenate([jnp.arange(0, C, 2), jnp.arange(1, C, 2)])
        w1u, b1u = w1[idx][:, idx], b1[idx]
        w2u, b2u = w2[idx][:, idx], b2[idx]
    else:
        w1u, b1u, w2u, b2u = w1, b1, w2, b2

    slab_bytes = rows * pl.cdiv(lanes, 128) * 128 * xv.dtype.itemsize
    group = max(1, min(B, (52 << 20) // (4 * slab_bytes)))
    while B % group:
        group -= 1
    n_steps = B // group

    vmem = 4 * group * slab_bytes + 2 * C * C * 4 + (4 << 20)

    out = pl.pallas_call(
        functools.partial(_se_kernel, hw=HW, pair=pair),
        out_shape=jax.ShapeDtypeStruct((B, rows, lanes), xv.dtype),
        grid=(n_steps,),
        in_specs=[
            pl.BlockSpec((group, rows, lanes), lambda i: (i, 0, 0)),
            pl.BlockSpec((C, C), lambda i: (0, 0)),
            pl.BlockSpec((1, C), lambda i: (0, 0)),
            pl.BlockSpec((C, C), lambda i: (0, 0)),
            pl.BlockSpec((1, C), lambda i: (0, 0)),
        ],
        out_specs=pl.BlockSpec((group, rows, lanes), lambda i: (i, 0, 0)),
        compiler_params=pltpu.CompilerParams(
            dimension_semantics=("arbitrary",),
            vmem_limit_bytes=int(min(vmem, 60 << 20))),
    )(xv, w1u.T, b1u.reshape(1, C), w2u.T, b2u.reshape(1, C))

    return out.reshape(B, C, H, W)
